# baseline (speedup 1.0000x reference)
import functools

import jax
import jax.numpy as jnp
from jax import lax
from jax.experimental import pallas as pl
from jax.experimental.pallas import tpu as pltpu
from jax.experimental.pallas import tpu_sc as plsc

N = 10000
E = 320000
D = 128
DP = 144
NC = 2
NS = 16
NW = NC * NS
EPW = E // NW
CHUNK = 64
NCHUNK = 160
EPWP = CHUNK * NCHUNK
NPADROWS = EPWP - EPW
NPAD = 10240
ZROWS = NPAD // NS


def _make_sc_kernel():
    mesh = plsc.VectorSubcoreMesh(core_axis_name="c", subcore_axis_name="s")

    @functools.partial(
        pl.kernel,
        out_type=jax.ShapeDtypeStruct((NC, NPAD, DP), jnp.float32),
        mesh=mesh,
        compiler_params=pltpu.CompilerParams(use_tc_tiling_on_sc=False),
        scratch_types=[
            pltpu.VMEM_SHARED((NPAD, DP), jnp.float32),
            pltpu.VMEM((CHUNK,), jnp.int32),
            pltpu.VMEM((CHUNK,), jnp.int32),
            pltpu.VMEM((CHUNK, DP), jnp.float32),
            pltpu.SemaphoreType.DMA,
        ],
    )
    def sc_kernel(xa_hbm, src_hbm, dst_hbm, zero_hbm, out_hbm,
                  acc, idx_s, idx_d, rows, gsem):
        core = lax.axis_index("c")
        sub = lax.axis_index("s")
        wid = core * NS + sub

        pltpu.sync_copy(zero_hbm, acc.at[pl.ds(sub * ZROWS, ZROWS)])
        plsc.subcore_barrier()

        ebase = wid * EPWP

        def chunk_body(c, carry):
            off = ebase + c * CHUNK
            pltpu.sync_copy(src_hbm.at[pl.ds(off, CHUNK)], idx_s)
            pltpu.sync_copy(dst_hbm.at[pl.ds(off, CHUNK)], idx_d)
            pltpu.async_copy(xa_hbm.at[idx_s], rows, gsem).wait()
            pltpu.sync_copy(rows, acc.at[idx_d], add=True)
            return carry

        lax.fori_loop(0, NCHUNK, chunk_body, 0)
        plsc.subcore_barrier()

        pltpu.sync_copy(acc.at[pl.ds(sub * ZROWS, ZROWS)],
                        out_hbm.at[core, pl.ds(sub * ZROWS, ZROWS)])

    return sc_kernel


_BN = 1000


def _tc_body(x_ref, p_ref, ws_ref, wn_ref, b_ref, o_ref):
    x = x_ref[...]
    s = p_ref[0] + p_ref[1]
    agg = s[:, :D]
    deg = s[:, D:D + 1]
    a = (agg + x) / (deg + 1.0)
    out = (jnp.dot(x, ws_ref[...], preferred_element_type=jnp.float32)
           + jnp.dot(a, wn_ref[...], preferred_element_type=jnp.float32)
           + b_ref[...])
    o_ref[...] = jnp.where(out >= 0, out, 0.01 * out)


def _tc_call(x, partials, W_self, W_neigh, b2d):
    grid = (N // _BN,)
    return pl.pallas_call(
        _tc_body,
        grid=grid,
        in_specs=[
            pl.BlockSpec((_BN, D), lambda i: (i, 0)),
            pl.BlockSpec((2, _BN, DP), lambda i: (0, i, 0)),
            pl.BlockSpec((D, D), lambda i: (0, 0)),
            pl.BlockSpec((D, D), lambda i: (0, 0)),
            pl.BlockSpec((1, D), lambda i: (0, 0)),
        ],
        out_specs=pl.BlockSpec((_BN, D), lambda i: (i, 0)),
        out_shape=jax.ShapeDtypeStruct((N, D), jnp.float32),
    )(x, partials, W_self, W_neigh, b2d)


def kernel(x, edge_index, W_self, W_neigh, b):
    ei = edge_index.astype(jnp.int32)
    src = ei[0].reshape(NW, EPW)
    dst = ei[1].reshape(NW, EPW)
    pad_src = jnp.zeros((NW, NPADROWS), jnp.int32)
    pad_dst = jnp.broadcast_to(
        (N + jnp.arange(NPADROWS, dtype=jnp.int32))[None, :], (NW, NPADROWS))
    srcp = jnp.concatenate([src, pad_src], axis=1).reshape(NW * EPWP)
    dstp = jnp.concatenate([dst, pad_dst], axis=1).reshape(NW * EPWP)
    xa = jnp.concatenate(
        [x,
         jnp.ones((N, 1), jnp.float32),
         jnp.zeros((N, DP - D - 1), jnp.float32)], axis=1)
    zero = jnp.zeros((ZROWS, DP), jnp.float32)
    partials = _make_sc_kernel()(xa, srcp, dstp, zero)
    return _tc_call(x, partials, W_self, W_neigh, b.reshape(1, D))


# speedup vs baseline: 1.0617x; 1.0617x over previous
import functools

import jax
import jax.numpy as jnp
from jax import lax
from jax.experimental import pallas as pl
from jax.experimental.pallas import tpu as pltpu
from jax.experimental.pallas import tpu_sc as plsc

N = 10000
E = 320000
D = 128
DP = 144
NC = 2
NS = 16
NW = NC * NS
EPW = E // NW
CHUNK = 80
NCHUNK = 128
EPWP = CHUNK * NCHUNK
NPADROWS = EPWP - EPW
NPAD = 10240
ZROWS = NPAD // NS


def _make_sc_kernel():
    mesh = plsc.VectorSubcoreMesh(core_axis_name="c", subcore_axis_name="s")

    @functools.partial(
        pl.kernel,
        out_type=jax.ShapeDtypeStruct((NC, NPAD, DP), jnp.float32),
        mesh=mesh,
        compiler_params=pltpu.CompilerParams(use_tc_tiling_on_sc=False),
        scratch_types=[
            pltpu.VMEM_SHARED((NPAD, DP), jnp.float32),
            pltpu.VMEM((CHUNK,), jnp.int32),
            pltpu.VMEM((CHUNK,), jnp.int32),
            pltpu.VMEM((CHUNK, DP), jnp.float32),
            pltpu.SemaphoreType.DMA,
        ],
    )
    def sc_kernel(xa_hbm, src_hbm, dst_hbm, zero_hbm, out_hbm,
                  acc, idx_s, idx_d, rows, gsem):
        core = lax.axis_index("c")
        sub = lax.axis_index("s")
        wid = core * NS + sub

        pltpu.sync_copy(zero_hbm, acc.at[pl.ds(sub * ZROWS, ZROWS)])
        plsc.subcore_barrier()

        ebase = wid * EPWP

        def chunk_body(c, carry):
            off = ebase + c * CHUNK
            pltpu.sync_copy(src_hbm.at[pl.ds(off, CHUNK)], idx_s)
            pltpu.sync_copy(dst_hbm.at[pl.ds(off, CHUNK)], idx_d)
            pltpu.async_copy(xa_hbm.at[idx_s], rows, gsem).wait()
            pltpu.sync_copy(rows, acc.at[idx_d], add=True)
            return carry

        lax.fori_loop(0, NCHUNK, chunk_body, 0)
        plsc.subcore_barrier()

        pltpu.sync_copy(acc.at[pl.ds(sub * ZROWS, ZROWS)],
                        out_hbm.at[core, pl.ds(sub * ZROWS, ZROWS)])

    return sc_kernel


_BN = 1000


def _tc_body(x_ref, p_ref, ws_ref, wn_ref, b_ref, o_ref):
    x = x_ref[...]
    s = p_ref[0] + p_ref[1]
    agg = s[:, :D]
    deg = s[:, D:D + 1]
    a = (agg + x) / (deg + 1.0)
    out = (jnp.dot(x, ws_ref[...], preferred_element_type=jnp.float32)
           + jnp.dot(a, wn_ref[...], preferred_element_type=jnp.float32)
           + b_ref[...])
    o_ref[...] = jnp.where(out >= 0, out, 0.01 * out)


def _tc_call(x, partials, W_self, W_neigh, b2d):
    grid = (N // _BN,)
    return pl.pallas_call(
        _tc_body,
        grid=grid,
        in_specs=[
            pl.BlockSpec((_BN, D), lambda i: (i, 0)),
            pl.BlockSpec((2, _BN, DP), lambda i: (0, i, 0)),
            pl.BlockSpec((D, D), lambda i: (0, 0)),
            pl.BlockSpec((D, D), lambda i: (0, 0)),
            pl.BlockSpec((1, D), lambda i: (0, 0)),
        ],
        out_specs=pl.BlockSpec((_BN, D), lambda i: (i, 0)),
        out_shape=jax.ShapeDtypeStruct((N, D), jnp.float32),
    )(x, partials, W_self, W_neigh, b2d)


def kernel(x, edge_index, W_self, W_neigh, b):
    ei = edge_index.astype(jnp.int32)
    src = ei[0].reshape(NW, EPW)
    dst = ei[1].reshape(NW, EPW)
    pad_src = jnp.zeros((NW, NPADROWS), jnp.int32)
    pad_dst = jnp.broadcast_to(
        (N + jnp.arange(NPADROWS, dtype=jnp.int32))[None, :], (NW, NPADROWS))
    srcp = jnp.concatenate([src, pad_src], axis=1).reshape(NW * EPWP)
    dstp = jnp.concatenate([dst, pad_dst], axis=1).reshape(NW * EPWP)
    xa = jnp.concatenate(
        [x,
         jnp.ones((N, 1), jnp.float32),
         jnp.zeros((N, DP - D - 1), jnp.float32)], axis=1)
    zero = jnp.zeros((ZROWS, DP), jnp.float32)
    partials = _make_sc_kernel()(xa, srcp, dstp, zero)
    return _tc_call(x, partials, W_self, W_neigh, b.reshape(1, D))


# speedup vs baseline: 2.9735x; 2.8007x over previous
import functools

import jax
import jax.numpy as jnp
from jax import lax
from jax.experimental import pallas as pl
from jax.experimental.pallas import tpu as pltpu
from jax.experimental.pallas import tpu_sc as plsc

N = 10000
E = 320000
D = 128
DP = 144
NC = 2
NS = 16
NW = NC * NS
EPW = E // NW
CHUNK = 80
NCHUNK = EPW // CHUNK
NPAD = 10240
ZROWS = NPAD // NS


def _make_sc_kernel():
    mesh = plsc.VectorSubcoreMesh(core_axis_name="c", subcore_axis_name="s")

    @functools.partial(
        pl.kernel,
        out_type=jax.ShapeDtypeStruct((NC, NPAD, DP), jnp.float32),
        mesh=mesh,
        compiler_params=pltpu.CompilerParams(use_tc_tiling_on_sc=False),
        scratch_types=[
            pltpu.VMEM_SHARED((NPAD, DP), jnp.float32),
            pltpu.VMEM((2, CHUNK), jnp.int32),
            pltpu.VMEM((2, CHUNK), jnp.int32),
            pltpu.VMEM((2, CHUNK, DP), jnp.float32),
            pltpu.SemaphoreType.DMA,
            pltpu.SemaphoreType.DMA,
        ],
    )
    def sc_kernel(xa_hbm, src_hbm, dst_hbm, zero_hbm, out_hbm,
                  acc, idx_s, idx_d, rows, gsem, isem):
        core = lax.axis_index("c")
        sub = lax.axis_index("s")
        wid = core * NS + sub

        pltpu.sync_copy(zero_hbm, acc.at[pl.ds(sub * ZROWS, ZROWS)])
        plsc.subcore_barrier()

        ebase = wid * EPW

        def idx_start(c, b):
            off = ebase + c * CHUNK
            pltpu.async_copy(src_hbm.at[pl.ds(off, CHUNK)], idx_s.at[b], isem)
            pltpu.async_copy(dst_hbm.at[pl.ds(off, CHUNK)], idx_d.at[b], isem)

        def idx_wait(c, b):
            off = ebase + c * CHUNK
            pltpu.make_async_copy(src_hbm.at[pl.ds(off, CHUNK)], idx_s.at[b],
                                  isem).wait()
            pltpu.make_async_copy(dst_hbm.at[pl.ds(off, CHUNK)], idx_d.at[b],
                                  isem).wait()

        def gather(b):
            pltpu.async_copy(xa_hbm.at[idx_s.at[b]], rows.at[b], gsem)

        def gwait(b):
            pltpu.make_async_copy(xa_hbm.at[idx_s.at[b]], rows.at[b],
                                  gsem).wait()

        def scatter(b):
            pltpu.sync_copy(rows.at[b], acc.at[idx_d.at[b]], add=True)

        off0 = ebase
        pltpu.sync_copy(src_hbm.at[pl.ds(off0, CHUNK)], idx_s.at[0])
        pltpu.sync_copy(dst_hbm.at[pl.ds(off0, CHUNK)], idx_d.at[0])
        idx_start(1, 1)
        gather(0)

        def pair_body(p, carry):
            j = 2 * p
            gwait(0)
            idx_wait(j + 1, 1)
            gather(1)
            scatter(0)
            idx_start(j + 2, 0)

            gwait(1)
            idx_wait(j + 2, 0)
            gather(0)
            scatter(1)

            @pl.when(p < NCHUNK // 2 - 1)
            def _():
                idx_start(j + 3, 1)
            return carry

        lax.fori_loop(0, NCHUNK // 2, pair_body, 0)

        gwait(0)
        scatter(0)
        plsc.subcore_barrier()

        pltpu.sync_copy(acc.at[pl.ds(sub * ZROWS, ZROWS)],
                        out_hbm.at[core, pl.ds(sub * ZROWS, ZROWS)])

    return sc_kernel


_BN = 1000


def _tc_body(x_ref, p_ref, ws_ref, wn_ref, b_ref, o_ref):
    x = x_ref[...]
    s = p_ref[0] + p_ref[1]
    agg = s[:, :D]
    deg = s[:, D:D + 1]
    a = (agg + x) / (deg + 1.0)
    out = (jnp.dot(x, ws_ref[...], preferred_element_type=jnp.float32)
           + jnp.dot(a, wn_ref[...], preferred_element_type=jnp.float32)
           + b_ref[...])
    o_ref[...] = jnp.where(out >= 0, out, 0.01 * out)


def _tc_call(x, partials, W_self, W_neigh, b2d):
    grid = (N // _BN,)
    return pl.pallas_call(
        _tc_body,
        grid=grid,
        in_specs=[
            pl.BlockSpec((_BN, D), lambda i: (i, 0)),
            pl.BlockSpec((2, _BN, DP), lambda i: (0, i, 0)),
            pl.BlockSpec((D, D), lambda i: (0, 0)),
            pl.BlockSpec((D, D), lambda i: (0, 0)),
            pl.BlockSpec((1, D), lambda i: (0, 0)),
        ],
        out_specs=pl.BlockSpec((_BN, D), lambda i: (i, 0)),
        out_shape=jax.ShapeDtypeStruct((N, D), jnp.float32),
    )(x, partials, W_self, W_neigh, b2d)


def kernel(x, edge_index, W_self, W_neigh, b):
    ei = edge_index.astype(jnp.int32)
    srcp = ei[0]
    dstp = ei[1]
    xa = jnp.concatenate(
        [x,
         jnp.ones((N, 1), jnp.float32),
         jnp.zeros((N, DP - D - 1), jnp.float32)], axis=1)
    zero = jnp.zeros((ZROWS, DP), jnp.float32)
    partials = _make_sc_kernel()(xa, srcp, dstp, zero)
    return _tc_call(x, partials, W_self, W_neigh, b.reshape(1, D))
